# baseline (speedup 1.0000x reference)
import functools

import jax
import jax.numpy as jnp
from jax import lax
from jax.experimental import pallas as pl
from jax.experimental.pallas import tpu as pltpu
from jax.experimental.pallas import tpu_sc as plsc

B, S, D, E, K = 4, 8192, 4096, 64, 8
T = B * S
ROUTE_SCALE = 2.5

NC, NS = 2, 16
NW = NC * NS
CHUNK = T // NW
SUB = 256
LANES = 16

BT = 512


def _mm_body(x_ref, wt_ref, out_ref):
    out_ref[...] = lax.dot_general(
        x_ref[...], wt_ref[...], (((1,), (0,)), ((), ())),
        preferred_element_type=jnp.float32,
        precision=lax.Precision.DEFAULT,
    )


def _tc_logits(x, wt):
    return pl.pallas_call(
        _mm_body,
        grid=(T // BT,),
        in_specs=[
            pl.BlockSpec((BT, D), lambda i: (i, 0)),
            pl.BlockSpec((D, E), lambda i: (0, 0)),
        ],
        out_specs=pl.BlockSpec((BT, E), lambda i: (i, 0)),
        out_shape=jax.ShapeDtypeStruct((T, E), jnp.float32),
    )(x, wt)


def _sc_router_body(logits_hbm, bias_hbm, out_hbm, logits_v, bias_v, out_v):
    wid = lax.axis_index("s") * NC + lax.axis_index("c")
    pltpu.sync_copy(bias_hbm, bias_v)

    iota = lax.iota(jnp.int32, LANES)
    mask8 = iota < K

    def merge(ak, av, bk, bv):
        rbk = lax.rev(bk, (0,))
        rbv = lax.rev(bv, (0,))
        m = ak >= rbk
        mk = jnp.where(m, ak, rbk)
        mv = jnp.where(m, av, rbv)
        return plsc.sort_key_val(mk, mv, descending=True)

    def row(r):
        ks, vs = [], []
        for i in range(E // LANES):
            l = logits_v[r, pl.ds(LANES * i, LANES)]
            s = 1.0 / (1.0 + jnp.exp(-l))
            b = s + bias_v[pl.ds(LANES * i, LANES)]
            k_, v_ = plsc.sort_key_val(b, iota + LANES * i, descending=True)
            ks.append(k_)
            vs.append(v_)
        k01, v01 = merge(ks[0], vs[0], ks[1], vs[1])
        k23, v23 = merge(ks[2], vs[2], ks[3], vs[3])
        kt, vt = merge(k01, v01, k23, v23)
        bsel = plsc.load_gather(bias_v, [vt])
        raw = kt - bsel
        denom = jnp.sum(jnp.where(mask8, raw, jnp.zeros_like(raw)))
        denom_v = lax.broadcast(denom + 1e-20, (LANES,))
        norm = (raw * ROUTE_SCALE) / denom_v
        vtf = plsc.bitcast(vt, jnp.float32)
        out_v[r, :] = jnp.where(mask8, norm, lax.rev(vtf, (0,)))

    def subchunk(j, carry):
        base = wid * CHUNK + j * SUB
        pltpu.sync_copy(logits_hbm.at[pl.ds(base, SUB)], logits_v)
        plsc.parallel_loop(0, SUB, unroll=4)(row)
        pltpu.sync_copy(out_v, out_hbm.at[pl.ds(base, SUB)])
        return carry

    lax.fori_loop(0, CHUNK // SUB, subchunk, 0)


@functools.partial(
    pl.kernel,
    out_type=jax.ShapeDtypeStruct((T, LANES), jnp.float32),
    mesh=plsc.VectorSubcoreMesh(core_axis_name="c", subcore_axis_name="s"),
    scratch_types=[
        pltpu.VMEM((SUB, E), jnp.float32),
        pltpu.VMEM((E,), jnp.float32),
        pltpu.VMEM((SUB, LANES), jnp.float32),
    ],
    compiler_params=pltpu.CompilerParams(
        needs_layout_passes=False, use_tc_tiling_on_sc=True),
)
def _sc_router(logits_hbm, bias_hbm, out_hbm, logits_v, bias_v, out_v):
    _sc_router_body(logits_hbm, bias_hbm, out_hbm, logits_v, bias_v, out_v)


def kernel(hidden_states, expert_bias, W):
    x = hidden_states.reshape(-1, D)
    logits = _tc_logits(x, W.T)
    packed = _sc_router(logits, expert_bias)
    top_scores = packed[:, :K]
    sel = lax.bitcast_convert_type(packed[:, K:LANES], jnp.int32)[:, ::-1]
    return logits, top_scores, sel


# speedup vs baseline: 1.5526x; 1.5526x over previous
import functools

import jax
import jax.numpy as jnp
from jax import lax
from jax.experimental import pallas as pl
from jax.experimental.pallas import tpu as pltpu
from jax.experimental.pallas import tpu_sc as plsc

B, S, D, E, K = 4, 8192, 4096, 64, 8
T = B * S
ROUTE_SCALE = 2.5

NC, NS = 2, 16
NW = NC * NS
CHUNK = T // NW
SUB = 256
LANES = 16

BT = 512


def _mm_body(x_ref, wt_ref, out_ref):
    out_ref[...] = lax.dot_general(
        x_ref[...], wt_ref[...], (((1,), (0,)), ((), ())),
        preferred_element_type=jnp.float32,
        precision=lax.Precision.DEFAULT,
    )


def _tc_logits(x, wt):
    return pl.pallas_call(
        _mm_body,
        grid=(T // BT,),
        in_specs=[
            pl.BlockSpec((BT, D), lambda i: (i, 0)),
            pl.BlockSpec((D, E), lambda i: (0, 0)),
        ],
        out_specs=pl.BlockSpec((BT, E), lambda i: (i, 0)),
        out_shape=jax.ShapeDtypeStruct((T, E), jnp.float32),
    )(x, wt)


def _sc_router_body(logits_hbm, bias_hbm, out_hbm, logits_v, bias_v, out_v):
    wid = lax.axis_index("s") * NC + lax.axis_index("c")
    pltpu.sync_copy(bias_hbm, bias_v)

    iota = lax.iota(jnp.int32, LANES)
    mask8 = iota < K

    def merge(ak, av, bk, bv):
        rbk = lax.rev(bk, (0,))
        rbv = lax.rev(bv, (0,))
        m = ak >= rbk
        mk = jnp.where(m, ak, rbk)
        mv = jnp.where(m, av, rbv)
        return plsc.sort_key_val(mk, mv, descending=True)

    def row(r):
        ks, vs = [], []
        for i in range(E // LANES):
            l = logits_v[r, pl.ds(LANES * i, LANES)]
            s = 1.0 / (1.0 + jnp.exp(-l))
            b = s + bias_v[pl.ds(LANES * i, LANES)]
            k_, v_ = plsc.sort_key_val(b, iota + LANES * i, descending=True)
            ks.append(k_)
            vs.append(v_)
        k01, v01 = merge(ks[0], vs[0], ks[1], vs[1])
        k23, v23 = merge(ks[2], vs[2], ks[3], vs[3])
        kt, vt = merge(k01, v01, k23, v23)
        bsel = plsc.load_gather(bias_v, [vt])
        raw = kt - bsel
        denom = jnp.sum(jnp.where(mask8, raw, jnp.zeros_like(raw)))
        denom_v = lax.broadcast(denom + 1e-20, (LANES,))
        norm = (raw * ROUTE_SCALE) / denom_v
        _, vt_sw = plsc.sort_key_val(jnp.bitwise_xor(iota, K), vt)
        out_v[r, :] = jnp.where(mask8, norm, plsc.bitcast(vt_sw, jnp.float32))

    def subchunk(j, carry):
        base = wid * CHUNK + j * SUB
        pltpu.sync_copy(logits_hbm.at[pl.ds(base, SUB)], logits_v)
        plsc.parallel_loop(0, SUB, unroll=4)(row)
        pltpu.sync_copy(out_v, out_hbm.at[pl.ds(base, SUB)])
        return carry

    lax.fori_loop(0, CHUNK // SUB, subchunk, 0)


@functools.partial(
    pl.kernel,
    out_type=jax.ShapeDtypeStruct((T, LANES), jnp.float32),
    mesh=plsc.VectorSubcoreMesh(core_axis_name="c", subcore_axis_name="s"),
    scratch_types=[
        pltpu.VMEM((SUB, E), jnp.float32),
        pltpu.VMEM((E,), jnp.float32),
        pltpu.VMEM((SUB, LANES), jnp.float32),
    ],
    compiler_params=pltpu.CompilerParams(
        needs_layout_passes=False, use_tc_tiling_on_sc=True),
)
def _sc_router(logits_hbm, bias_hbm, out_hbm, logits_v, bias_v, out_v):
    _sc_router_body(logits_hbm, bias_hbm, out_hbm, logits_v, bias_v, out_v)


def kernel(hidden_states, expert_bias, W):
    x = hidden_states.reshape(-1, D)
    logits = _tc_logits(x, W.T)
    packed = _sc_router(logits, expert_bias)
    top_scores = packed[:, :K]
    sel = lax.bitcast_convert_type(packed[:, K:LANES], jnp.int32)
    return logits, top_scores, sel
